# baseline (speedup 1.0000x reference)
import functools

import jax
import jax.numpy as jnp
from jax import lax
from jax.experimental import pallas as pl
from jax.experimental.pallas import tpu as pltpu
from jax.experimental.pallas import tpu_sc as plsc

N = 10000
NP = 10240
E = 320000
G = 64
NC = 2
NS = 16
ROWS_PER_TILE = NP // NS
CHUNK = 128
RB = NP // 128


def _sc_mesh():
    return plsc.VectorSubcoreMesh(
        core_axis_name="c", subcore_axis_name="s",
        num_cores=NC, num_subcores=NS)


_EPT32 = E // (NC * NS)
_DEG_CHUNKS = _EPT32 // CHUNK
_DEG_TAIL = _EPT32 - _DEG_CHUNKS * CHUNK


def _degree_fn():
    @functools.partial(
        pl.kernel,
        out_type=jax.ShapeDtypeStruct((NC, NP, 16), jnp.float32),
        mesh=_sc_mesh(),
        compiler_params=pltpu.CompilerParams(use_tc_tiling_on_sc=False),
        scratch_types=[
            pltpu.VMEM_SHARED((NP, 16), jnp.float32),
            pltpu.VMEM((CHUNK, 16), jnp.float32),
            pltpu.VMEM((CHUNK,), jnp.int32),
            pltpu.VMEM((_DEG_TAIL, 16), jnp.float32),
            pltpu.VMEM((_DEG_TAIL,), jnp.int32),
        ],
    )
    def deg_kernel(dst_hbm, zeros_hbm, ones_hbm, out_hbm,
                   deg_sp, ones_v, idx_v, ones_t, idx_t):
        c = lax.axis_index("c")
        s = lax.axis_index("s")
        r0 = s * ROWS_PER_TILE
        pltpu.sync_copy(zeros_hbm.at[pl.ds(r0, ROWS_PER_TILE)],
                        deg_sp.at[pl.ds(r0, ROWS_PER_TILE)])
        pltpu.sync_copy(ones_hbm, ones_v)
        pltpu.sync_copy(ones_hbm.at[pl.ds(0, _DEG_TAIL)], ones_t)
        plsc.subcore_barrier()
        base = (c * NS + s) * _EPT32

        def chunk(j, carry):
            off = pl.multiple_of(base + j * CHUNK, 8)
            pltpu.sync_copy(dst_hbm.at[pl.ds(off, CHUNK)], idx_v)
            pltpu.sync_copy(ones_v, deg_sp.at[idx_v], add=True)
            return carry

        lax.fori_loop(0, _DEG_CHUNKS, chunk, 0)
        toff = pl.multiple_of(base + _DEG_CHUNKS * CHUNK, 8)
        pltpu.sync_copy(dst_hbm.at[pl.ds(toff, _DEG_TAIL)], idx_t)
        pltpu.sync_copy(ones_t, deg_sp.at[idx_t], add=True)
        plsc.subcore_barrier()
        pltpu.sync_copy(deg_sp.at[pl.ds(r0, ROWS_PER_TILE)],
                        out_hbm.at[c, pl.ds(r0, ROWS_PER_TILE)])

    return deg_kernel


EPT = 20480
EPAD = NS * EPT


def _make_spmm(dh, chunk, depth):
    nchunk = EPT // chunk
    ah = depth // 2
    ng = nchunk // depth
    nsub = chunk // 16

    @functools.partial(
        pl.kernel,
        out_type=jax.ShapeDtypeStruct((NC * NP, dh), jnp.float32),
        mesh=_sc_mesh(),
        compiler_params=pltpu.CompilerParams(use_tc_tiling_on_sc=False),
        scratch_types=[
            pltpu.VMEM_SHARED((NP, dh), jnp.float32),
            pltpu.VMEM((nchunk, chunk), jnp.int32),
        ] + [pltpu.VMEM((chunk, dh), jnp.float32)] * depth
          + [pltpu.VMEM((chunk,), jnp.int32)] * (2 * depth)
          + [pltpu.SemaphoreType.DMA] * (2 * depth),
    )
    def spmm(g_hbm, pk_hbm, out_hbm, s_sp, pk_all, *bufs):
        rows = bufs[0:depth]
        src_u = bufs[depth:2 * depth]
        dst_u = bufs[2 * depth:3 * depth]
        gsem = bufs[3 * depth:4 * depth]
        ssem = bufs[4 * depth:5 * depth]
        c = lax.axis_index("c")
        s = lax.axis_index("s")
        r0 = s * ROWS_PER_TILE
        tbl = c * NP
        pltpu.sync_copy(g_hbm.at[pl.ds(tbl + r0, ROWS_PER_TILE)],
                        s_sp.at[pl.ds(r0, ROWS_PER_TILE)])
        pltpu.sync_copy(pk_hbm.at[c, s], pk_all)
        plsc.subcore_barrier()

        def unpack(j, b):
            for k in range(nsub):
                p = pk_all[j, pl.ds(k * 16, 16)]
                src_u[b][pl.ds(k * 16, 16)] = p >> 15
                dst_u[b][pl.ds(k * 16, 16)] = p & 0x7FFF

        def start_gather(j, b):
            pltpu.async_copy(g_hbm.at[src_u[b]], rows[b], gsem[b])

        def start_scatter(b):
            pltpu.async_copy(rows[b], s_sp.at[dst_u[b]], ssem[b], add=True)

        def drain(sem, b):
            pltpu.make_async_copy(g_hbm.at[pl.ds(0, chunk)], rows[b],
                                  sem).wait()

        for j in range(ah):
            unpack(j, j)
            start_gather(j, j)

        def group(g, carry):
            for b in range(depth):
                j = g * depth + b
                drain(gsem[b], b)
                start_scatter(b)
                bp = (b + ah) % depth
                if b < ah:
                    @pl.when(g >= 1)
                    def _():
                        drain(ssem[bp], bp)
                    unpack(j + ah, bp)
                    start_gather(j + ah, bp)
                else:
                    @pl.when(g < ng - 1)
                    def _():
                        drain(ssem[bp], bp)
                        unpack(j + ah, bp)
                        start_gather(j + ah, bp)
            return carry

        lax.fori_loop(0, ng, group, 0)
        for b in range(depth):
            drain(ssem[b], b)
        plsc.subcore_barrier()
        pltpu.sync_copy(s_sp.at[pl.ds(r0, ROWS_PER_TILE)],
                        out_hbm.at[pl.ds(tbl + r0, ROWS_PER_TILE)])

    return spmm


def _prep_body(deg_ref, x_ref, dinv_ref, g_ref):
    deg = deg_ref[0, :, 0:1] + deg_ref[1, :, 0:1] + 1.0
    dinv = lax.rsqrt(deg)
    dinv_ref[...] = dinv
    g = x_ref[...] * dinv
    g_ref[0] = g[:, 0:64]
    g_ref[1] = g[:, 64:128]


def _prep(deg, x):
    return pl.pallas_call(
        _prep_body,
        grid=(RB,),
        in_specs=[
            pl.BlockSpec((2, 128, 16), lambda r: (0, r, 0)),
            pl.BlockSpec((128, 128), lambda r: (r, 0)),
        ],
        out_specs=[
            pl.BlockSpec((128, 1), lambda r: (r, 0)),
            pl.BlockSpec((2, 128, 64), lambda r: (0, r, 0)),
        ],
        out_shape=[
            jax.ShapeDtypeStruct((NP, 1), jnp.float32),
            jax.ShapeDtypeStruct((2, NP, 64), jnp.float32),
        ],
        compiler_params=pltpu.CompilerParams(
            dimension_semantics=("arbitrary",)),
    )(deg, x)


def _make_layer(d_in, d_out):
    dh_in = d_in // 2
    dh_out = d_out // 2

    def body(s_ref, dinv_ref, w_ref, b_ref, out_ref):
        t = jnp.dot(s_ref[0], w_ref[0:dh_in, :],
                    preferred_element_type=jnp.float32)
        t = t + jnp.dot(s_ref[1], w_ref[dh_in:d_in, :],
                        preferred_element_type=jnp.float32)
        act = jnp.maximum(dinv_ref[...] * t + b_ref[...], 0.0)
        g = act * dinv_ref[...]
        rid = lax.broadcasted_iota(jnp.int32, (128, 1), 0) + pl.program_id(0) * 128
        g = jnp.where(rid < N, g, 0.0)
        out_ref[0] = g[:, 0:dh_out]
        out_ref[1] = g[:, dh_out:d_out]

    def layer(s, dinv, w, b):
        return pl.pallas_call(
            body,
            grid=(RB,),
            in_specs=[
                pl.BlockSpec((2, 128, dh_in), lambda r: (0, r, 0)),
                pl.BlockSpec((128, 1), lambda r: (r, 0)),
                pl.BlockSpec((d_in, d_out), lambda r: (0, 0)),
                pl.BlockSpec((1, d_out), lambda r: (0, 0)),
            ],
            out_specs=pl.BlockSpec((2, 128, dh_out), lambda r: (0, r, 0)),
            out_shape=jax.ShapeDtypeStruct((2, NP, dh_out), jnp.float32),
            compiler_params=pltpu.CompilerParams(
                dimension_semantics=("arbitrary",)),
        )(s, dinv, w, b)

    return layer


def _pool_body(s_ref, dinv_ref, batch_ref, w_ref, b_ref, wfc_ref, bfc_ref,
               out_ref, acc_ref):
    r = pl.program_id(0)

    @pl.when(r == 0)
    def _init():
        acc_ref[...] = jnp.full((G, 512), -jnp.inf, jnp.float32)

    t = jnp.dot(s_ref[0], w_ref[0:128, :], preferred_element_type=jnp.float32)
    t = t + jnp.dot(s_ref[1], w_ref[128:256, :],
                    preferred_element_type=jnp.float32)
    h = jnp.maximum(dinv_ref[...] * t + b_ref[...], 0.0)
    rid = lax.broadcasted_iota(jnp.int32, (128, 1), 0) + r * 128
    hm = jnp.where(rid < N, h, -jnp.inf)
    g_lo = jnp.clip(batch_ref[0, 0], 0, G - 1)
    g_hi = jnp.clip(batch_ref[127, 0], g_lo, G - 1)
    gids = lax.broadcasted_iota(jnp.int32, (G, 1), 0)

    def body(g, carry):
        sel = jnp.where(batch_ref[...] == g, hm, -jnp.inf)
        colmax = jnp.max(sel, axis=0, keepdims=True)
        acc_ref[...] = jnp.where(gids == g,
                                 jnp.maximum(acc_ref[...], colmax),
                                 acc_ref[...])
        return carry

    lax.fori_loop(g_lo, g_hi + 1, body, 0)

    @pl.when(r == RB - 1)
    def _fin():
        out_ref[...] = jnp.dot(acc_ref[...], wfc_ref[...],
                               preferred_element_type=jnp.float32) + bfc_ref[...]


def _pool(s3, dinv, batch2, w3, b3, wfc, bfc):
    return pl.pallas_call(
        _pool_body,
        grid=(RB,),
        in_specs=[
            pl.BlockSpec((2, 128, 128), lambda r: (0, r, 0)),
            pl.BlockSpec((128, 1), lambda r: (r, 0)),
            pl.BlockSpec((128, 1), lambda r: (r, 0)),
            pl.BlockSpec((256, 512), lambda r: (0, 0)),
            pl.BlockSpec((1, 512), lambda r: (0, 0)),
            pl.BlockSpec((512, 128), lambda r: (0, 0)),
            pl.BlockSpec((1, 128), lambda r: (0, 0)),
        ],
        out_specs=pl.BlockSpec((G, 128), lambda r: (0, 0)),
        out_shape=jax.ShapeDtypeStruct((G, 128), jnp.float32),
        scratch_shapes=[pltpu.VMEM((G, 512), jnp.float32)],
        compiler_params=pltpu.CompilerParams(
            dimension_semantics=("arbitrary",)),
    )(s3, dinv, batch2, w3, b3, wfc, bfc)


_degree = _degree_fn()
_spmm64 = _make_spmm(64, 128, 4)
_spmm128 = _make_spmm(128, 80, 2)
_layer1 = _make_layer(128, 128)
_layer2 = _make_layer(128, 256)


def kernel(x, edge_index, batch, W1, b1, W2, b2, W3, b3, Wfc, bfc):
    src = edge_index[0].astype(jnp.int32)
    dst = edge_index[1].astype(jnp.int32)
    srcp = jnp.concatenate([src, jnp.full((EPAD - E,), N, jnp.int32)])
    dstp = jnp.concatenate([dst, jnp.zeros((EPAD - E,), jnp.int32)])
    pkflat = jnp.stack([(srcp << 15) | dstp,
                        ((srcp + NP) << 15) | dstp])
    pk64 = pkflat.reshape(NC, NS, EPT // 128, 128)
    pk128 = pkflat.reshape(NC, NS, EPT // 80, 80)
    zeros16 = jnp.zeros((NP, 16), jnp.float32)
    ones16 = jnp.ones((CHUNK, 16), jnp.float32)
    x_pad = jnp.concatenate(
        [x, jnp.zeros((NP - N, x.shape[1]), jnp.float32)])
    batch2 = jnp.concatenate(
        [batch.astype(jnp.int32),
         jnp.full((NP - N,), G - 1, jnp.int32)]).reshape(NP, 1)

    deg = _degree(dst, zeros16, ones16)
    dinv, g1 = _prep(deg, x_pad)
    s1 = _spmm64(g1.reshape(2 * NP, 64), pk64).reshape(2, NP, 64)
    g2 = _layer1(s1, dinv, W1, b1.reshape(1, -1))
    s2 = _spmm64(g2.reshape(2 * NP, 64), pk64).reshape(2, NP, 64)
    g3 = _layer2(s2, dinv, W2, b2.reshape(1, -1))
    s3 = _spmm128(g3.reshape(2 * NP, 128), pk128).reshape(2, NP, 128)
    out = _pool(s3, dinv, batch2, W3, b3.reshape(1, -1),
                Wfc, bfc.reshape(1, -1))
    return out


# speedup vs baseline: 1.0333x; 1.0333x over previous
import functools

import jax
import jax.numpy as jnp
from jax import lax
from jax.experimental import pallas as pl
from jax.experimental.pallas import tpu as pltpu
from jax.experimental.pallas import tpu_sc as plsc

N = 10000
NP = 10240
E = 320000
G = 64
NC = 2
NS = 16
ROWS_PER_TILE = NP // NS
CHUNK = 128
RB = NP // 128


def _sc_mesh():
    return plsc.VectorSubcoreMesh(
        core_axis_name="c", subcore_axis_name="s",
        num_cores=NC, num_subcores=NS)


_EPT32 = E // (NC * NS)
_DEG_CHUNKS = _EPT32 // CHUNK
_DEG_TAIL = _EPT32 - _DEG_CHUNKS * CHUNK


def _degree_fn():
    @functools.partial(
        pl.kernel,
        out_type=jax.ShapeDtypeStruct((NC, NP, 16), jnp.float32),
        mesh=_sc_mesh(),
        compiler_params=pltpu.CompilerParams(use_tc_tiling_on_sc=False),
        scratch_types=[
            pltpu.VMEM_SHARED((NP, 16), jnp.float32),
            pltpu.VMEM((CHUNK, 16), jnp.float32),
            pltpu.VMEM((CHUNK,), jnp.int32),
            pltpu.VMEM((_DEG_TAIL, 16), jnp.float32),
            pltpu.VMEM((_DEG_TAIL,), jnp.int32),
        ],
    )
    def deg_kernel(dst_hbm, zeros_hbm, ones_hbm, out_hbm,
                   deg_sp, ones_v, idx_v, ones_t, idx_t):
        c = lax.axis_index("c")
        s = lax.axis_index("s")
        r0 = s * ROWS_PER_TILE
        pltpu.sync_copy(zeros_hbm.at[pl.ds(r0, ROWS_PER_TILE)],
                        deg_sp.at[pl.ds(r0, ROWS_PER_TILE)])
        pltpu.sync_copy(ones_hbm, ones_v)
        pltpu.sync_copy(ones_hbm.at[pl.ds(0, _DEG_TAIL)], ones_t)
        plsc.subcore_barrier()
        base = (c * NS + s) * _EPT32

        def chunk(j, carry):
            off = pl.multiple_of(base + j * CHUNK, 8)
            pltpu.sync_copy(dst_hbm.at[pl.ds(off, CHUNK)], idx_v)
            pltpu.sync_copy(ones_v, deg_sp.at[idx_v], add=True)
            return carry

        lax.fori_loop(0, _DEG_CHUNKS, chunk, 0)
        toff = pl.multiple_of(base + _DEG_CHUNKS * CHUNK, 8)
        pltpu.sync_copy(dst_hbm.at[pl.ds(toff, _DEG_TAIL)], idx_t)
        pltpu.sync_copy(ones_t, deg_sp.at[idx_t], add=True)
        plsc.subcore_barrier()
        pltpu.sync_copy(deg_sp.at[pl.ds(r0, ROWS_PER_TILE)],
                        out_hbm.at[c, pl.ds(r0, ROWS_PER_TILE)])

    return deg_kernel


EPT = 20480
EPAD = NS * EPT


def _make_spmm(dh, chunk, depth, phases):
    nchunk = EPT // chunk
    nchunk_p = nchunk // phases
    ah = depth // 2
    ng = nchunk_p // depth
    nsub = chunk // 16

    @functools.partial(
        pl.kernel,
        out_type=jax.ShapeDtypeStruct((NC * NP, dh), jnp.float32),
        mesh=_sc_mesh(),
        compiler_params=pltpu.CompilerParams(use_tc_tiling_on_sc=False),
        scratch_types=[
            pltpu.VMEM_SHARED((NP, dh), jnp.float32),
            pltpu.VMEM((nchunk_p, chunk), jnp.int32),
        ] + [pltpu.VMEM((chunk, dh), jnp.float32)] * depth
          + [pltpu.VMEM((chunk,), jnp.int32)] * (2 * depth)
          + [pltpu.SemaphoreType.DMA] * (2 * depth),
    )
    def spmm(g_hbm, pk_hbm, out_hbm, s_sp, pk_all, *bufs):
        rows = bufs[0:depth]
        src_u = bufs[depth:2 * depth]
        dst_u = bufs[2 * depth:3 * depth]
        gsem = bufs[3 * depth:4 * depth]
        ssem = bufs[4 * depth:5 * depth]
        c = lax.axis_index("c")
        s = lax.axis_index("s")
        r0 = s * ROWS_PER_TILE
        tbl = c * NP
        pltpu.sync_copy(g_hbm.at[pl.ds(tbl + r0, ROWS_PER_TILE)],
                        s_sp.at[pl.ds(r0, ROWS_PER_TILE)])
        plsc.subcore_barrier()

        def unpack(j, b):
            for k in range(nsub):
                p = pk_all[j, pl.ds(k * 16, 16)]
                src_u[b][pl.ds(k * 16, 16)] = p >> 15
                dst_u[b][pl.ds(k * 16, 16)] = p & 0x7FFF

        def start_gather(j, b):
            pltpu.async_copy(g_hbm.at[src_u[b]], rows[b], gsem[b])

        def start_scatter(b):
            pltpu.async_copy(rows[b], s_sp.at[dst_u[b]], ssem[b], add=True)

        def drain(sem, b):
            pltpu.make_async_copy(g_hbm.at[pl.ds(0, chunk)], rows[b],
                                  sem).wait()

        for ph in range(phases):
            pltpu.sync_copy(pk_hbm.at[c, s, pl.ds(ph * nchunk_p, nchunk_p)],
                            pk_all)
            for j in range(ah):
                unpack(j, j)
                start_gather(j, j)

            def group(g, carry):
                for b in range(depth):
                    j = g * depth + b
                    drain(gsem[b], b)
                    start_scatter(b)
                    bp = (b + ah) % depth
                    if b < ah:
                        @pl.when(g >= 1)
                        def _():
                            drain(ssem[bp], bp)
                        unpack(j + ah, bp)
                        start_gather(j + ah, bp)
                    else:
                        @pl.when(g < ng - 1)
                        def _():
                            drain(ssem[bp], bp)
                            unpack(j + ah, bp)
                            start_gather(j + ah, bp)
                return carry

            lax.fori_loop(0, ng, group, 0)
            for b in range(depth):
                drain(ssem[b], b)
        plsc.subcore_barrier()
        pltpu.sync_copy(s_sp.at[pl.ds(r0, ROWS_PER_TILE)],
                        out_hbm.at[pl.ds(tbl + r0, ROWS_PER_TILE)])

    return spmm


def _prep_body(deg_ref, x_ref, dinv_ref, g_ref):
    deg = deg_ref[0, :, 0:1] + deg_ref[1, :, 0:1] + 1.0
    dinv = lax.rsqrt(deg)
    dinv_ref[...] = dinv
    g = x_ref[...] * dinv
    g_ref[0] = g[:, 0:64]
    g_ref[1] = g[:, 64:128]


def _prep(deg, x):
    return pl.pallas_call(
        _prep_body,
        grid=(RB,),
        in_specs=[
            pl.BlockSpec((2, 128, 16), lambda r: (0, r, 0)),
            pl.BlockSpec((128, 128), lambda r: (r, 0)),
        ],
        out_specs=[
            pl.BlockSpec((128, 1), lambda r: (r, 0)),
            pl.BlockSpec((2, 128, 64), lambda r: (0, r, 0)),
        ],
        out_shape=[
            jax.ShapeDtypeStruct((NP, 1), jnp.float32),
            jax.ShapeDtypeStruct((2, NP, 64), jnp.float32),
        ],
        compiler_params=pltpu.CompilerParams(
            dimension_semantics=("arbitrary",)),
    )(deg, x)


def _make_layer(d_in, d_out):
    dh_in = d_in // 2
    dh_out = d_out // 2

    def body(s_ref, dinv_ref, w_ref, b_ref, out_ref):
        t = jnp.dot(s_ref[0], w_ref[0:dh_in, :],
                    preferred_element_type=jnp.float32)
        t = t + jnp.dot(s_ref[1], w_ref[dh_in:d_in, :],
                        preferred_element_type=jnp.float32)
        act = jnp.maximum(dinv_ref[...] * t + b_ref[...], 0.0)
        g = act * dinv_ref[...]
        rid = lax.broadcasted_iota(jnp.int32, (128, 1), 0) + pl.program_id(0) * 128
        g = jnp.where(rid < N, g, 0.0)
        out_ref[0] = g[:, 0:dh_out]
        out_ref[1] = g[:, dh_out:d_out]

    def layer(s, dinv, w, b):
        return pl.pallas_call(
            body,
            grid=(RB,),
            in_specs=[
                pl.BlockSpec((2, 128, dh_in), lambda r: (0, r, 0)),
                pl.BlockSpec((128, 1), lambda r: (r, 0)),
                pl.BlockSpec((d_in, d_out), lambda r: (0, 0)),
                pl.BlockSpec((1, d_out), lambda r: (0, 0)),
            ],
            out_specs=pl.BlockSpec((2, 128, dh_out), lambda r: (0, r, 0)),
            out_shape=jax.ShapeDtypeStruct((2, NP, dh_out), jnp.float32),
            compiler_params=pltpu.CompilerParams(
                dimension_semantics=("arbitrary",)),
        )(s, dinv, w, b)

    return layer


def _pool_body(s_ref, dinv_ref, batch_ref, w_ref, b_ref, wfc_ref, bfc_ref,
               out_ref, acc_ref):
    r = pl.program_id(0)

    @pl.when(r == 0)
    def _init():
        acc_ref[...] = jnp.full((G, 512), -jnp.inf, jnp.float32)

    t = jnp.dot(s_ref[0], w_ref[0:128, :], preferred_element_type=jnp.float32)
    t = t + jnp.dot(s_ref[1], w_ref[128:256, :],
                    preferred_element_type=jnp.float32)
    h = jnp.maximum(dinv_ref[...] * t + b_ref[...], 0.0)
    rid = lax.broadcasted_iota(jnp.int32, (128, 1), 0) + r * 128
    hm = jnp.where(rid < N, h, -jnp.inf)
    g_lo = jnp.clip(batch_ref[0, 0], 0, G - 1)
    g_hi = jnp.clip(batch_ref[127, 0], g_lo, G - 1)
    gids = lax.broadcasted_iota(jnp.int32, (G, 1), 0)

    def body(g, carry):
        sel = jnp.where(batch_ref[...] == g, hm, -jnp.inf)
        colmax = jnp.max(sel, axis=0, keepdims=True)
        acc_ref[...] = jnp.where(gids == g,
                                 jnp.maximum(acc_ref[...], colmax),
                                 acc_ref[...])
        return carry

    lax.fori_loop(g_lo, g_hi + 1, body, 0)

    @pl.when(r == RB - 1)
    def _fin():
        out_ref[...] = jnp.dot(acc_ref[...], wfc_ref[...],
                               preferred_element_type=jnp.float32) + bfc_ref[...]


def _pool(s3, dinv, batch2, w3, b3, wfc, bfc):
    return pl.pallas_call(
        _pool_body,
        grid=(RB,),
        in_specs=[
            pl.BlockSpec((2, 128, 128), lambda r: (0, r, 0)),
            pl.BlockSpec((128, 1), lambda r: (r, 0)),
            pl.BlockSpec((128, 1), lambda r: (r, 0)),
            pl.BlockSpec((256, 512), lambda r: (0, 0)),
            pl.BlockSpec((1, 512), lambda r: (0, 0)),
            pl.BlockSpec((512, 128), lambda r: (0, 0)),
            pl.BlockSpec((1, 128), lambda r: (0, 0)),
        ],
        out_specs=pl.BlockSpec((G, 128), lambda r: (0, 0)),
        out_shape=jax.ShapeDtypeStruct((G, 128), jnp.float32),
        scratch_shapes=[pltpu.VMEM((G, 512), jnp.float32)],
        compiler_params=pltpu.CompilerParams(
            dimension_semantics=("arbitrary",)),
    )(s3, dinv, batch2, w3, b3, wfc, bfc)


_degree = _degree_fn()
_spmm64 = _make_spmm(64, 128, 8, 1)
_spmm128 = _make_spmm(128, 128, 2, 2)
_layer1 = _make_layer(128, 128)
_layer2 = _make_layer(128, 256)


def kernel(x, edge_index, batch, W1, b1, W2, b2, W3, b3, Wfc, bfc):
    src = edge_index[0].astype(jnp.int32)
    dst = edge_index[1].astype(jnp.int32)
    srcp = jnp.concatenate([src, jnp.full((EPAD - E,), N, jnp.int32)])
    dstp = jnp.concatenate([dst, jnp.zeros((EPAD - E,), jnp.int32)])
    pkflat = jnp.stack([(srcp << 15) | dstp,
                        ((srcp + NP) << 15) | dstp])
    pk64 = pkflat.reshape(NC, NS, EPT // 128, 128)
    pk128 = pk64
    zeros16 = jnp.zeros((NP, 16), jnp.float32)
    ones16 = jnp.ones((CHUNK, 16), jnp.float32)
    x_pad = jnp.concatenate(
        [x, jnp.zeros((NP - N, x.shape[1]), jnp.float32)])
    batch2 = jnp.concatenate(
        [batch.astype(jnp.int32),
         jnp.full((NP - N,), G - 1, jnp.int32)]).reshape(NP, 1)

    deg = _degree(dst, zeros16, ones16)
    dinv, g1 = _prep(deg, x_pad)
    s1 = _spmm64(g1.reshape(2 * NP, 64), pk64).reshape(2, NP, 64)
    g2 = _layer1(s1, dinv, W1, b1.reshape(1, -1))
    s2 = _spmm64(g2.reshape(2 * NP, 64), pk64).reshape(2, NP, 64)
    g3 = _layer2(s2, dinv, W2, b2.reshape(1, -1))
    s3 = _spmm128(g3.reshape(2 * NP, 128), pk128).reshape(2, NP, 128)
    out = _pool(s3, dinv, batch2, W3, b3.reshape(1, -1),
                Wfc, bfc.reshape(1, -1))
    return out


# speedup vs baseline: 1.1056x; 1.0700x over previous
import functools

import jax
import jax.numpy as jnp
from jax import lax
from jax.experimental import pallas as pl
from jax.experimental.pallas import tpu as pltpu
from jax.experimental.pallas import tpu_sc as plsc

N = 10000
NP = 10240
E = 320000
G = 64
NC = 2
NS = 16
ROWS_PER_TILE = NP // NS
CHUNK = 128
RB = NP // 128


def _sc_mesh():
    return plsc.VectorSubcoreMesh(
        core_axis_name="c", subcore_axis_name="s",
        num_cores=NC, num_subcores=NS)


_EPT32 = E // (NC * NS)
_DEG_CHUNKS = _EPT32 // CHUNK
_DEG_TAIL = _EPT32 - _DEG_CHUNKS * CHUNK


def _degree_fn():
    @functools.partial(
        pl.kernel,
        out_type=jax.ShapeDtypeStruct((NC, NP, 16), jnp.float32),
        mesh=_sc_mesh(),
        compiler_params=pltpu.CompilerParams(use_tc_tiling_on_sc=False),
        scratch_types=[
            pltpu.VMEM_SHARED((NP, 16), jnp.float32),
            pltpu.VMEM((CHUNK, 16), jnp.float32),
            pltpu.VMEM((CHUNK,), jnp.int32),
            pltpu.VMEM((_DEG_TAIL, 16), jnp.float32),
            pltpu.VMEM((_DEG_TAIL,), jnp.int32),
        ],
    )
    def deg_kernel(dst_hbm, zeros_hbm, ones_hbm, out_hbm,
                   deg_sp, ones_v, idx_v, ones_t, idx_t):
        c = lax.axis_index("c")
        s = lax.axis_index("s")
        r0 = s * ROWS_PER_TILE
        pltpu.sync_copy(zeros_hbm.at[pl.ds(r0, ROWS_PER_TILE)],
                        deg_sp.at[pl.ds(r0, ROWS_PER_TILE)])
        pltpu.sync_copy(ones_hbm, ones_v)
        pltpu.sync_copy(ones_hbm.at[pl.ds(0, _DEG_TAIL)], ones_t)
        plsc.subcore_barrier()
        base = (c * NS + s) * _EPT32

        def chunk(j, carry):
            off = pl.multiple_of(base + j * CHUNK, 8)
            pltpu.sync_copy(dst_hbm.at[pl.ds(off, CHUNK)], idx_v)
            pltpu.sync_copy(ones_v, deg_sp.at[idx_v], add=True)
            return carry

        lax.fori_loop(0, _DEG_CHUNKS, chunk, 0)
        toff = pl.multiple_of(base + _DEG_CHUNKS * CHUNK, 8)
        pltpu.sync_copy(dst_hbm.at[pl.ds(toff, _DEG_TAIL)], idx_t)
        pltpu.sync_copy(ones_t, deg_sp.at[idx_t], add=True)
        plsc.subcore_barrier()
        pltpu.sync_copy(deg_sp.at[pl.ds(r0, ROWS_PER_TILE)],
                        out_hbm.at[c, pl.ds(r0, ROWS_PER_TILE)])

    return deg_kernel


EPT = 20480
EPAD = NS * EPT
DH = 64
SCHUNK = 128
IBLK = 10240


def _make_spmm(passes):
    nchunk = EPT // SCHUNK

    @functools.partial(
        pl.kernel,
        out_type=jax.ShapeDtypeStruct((NC * passes * NP, DH), jnp.float32),
        mesh=_sc_mesh(),
        compiler_params=pltpu.CompilerParams(use_tc_tiling_on_sc=False),
        scratch_types=[
            pltpu.VMEM_SHARED((NP, DH), jnp.float32),
            pltpu.VMEM_SHARED((NP, DH), jnp.float32),
            pltpu.VMEM((IBLK,), jnp.int32),
            pltpu.VMEM((IBLK,), jnp.int32),
            pltpu.VMEM((SCHUNK, DH), jnp.float32),
            pltpu.VMEM((SCHUNK, DH), jnp.float32),
            pltpu.SemaphoreType.DMA,
            pltpu.SemaphoreType.DMA,
        ],
    )
    def spmm(g_hbm, src_hbm, dst_hbm, out_hbm,
             tab_sp, acc_sp, src_v, dst_v, rows0, rows1, sem0, sem1):
        rows = (rows0, rows1)
        sems = (sem0, sem1)
        c = lax.axis_index("c")
        s = lax.axis_index("s")
        r0 = s * ROWS_PER_TILE

        def start_gather(j, b):
            pltpu.async_copy(tab_sp.at[src_v.at[pl.ds(j * SCHUNK, SCHUNK)]],
                             rows[b], sems[b])

        def wait_gather(b):
            pltpu.make_async_copy(
                tab_sp.at[pl.ds(0, SCHUNK)], rows[b], sems[b]).wait()

        def scatter(j, b):
            pltpu.sync_copy(rows[b],
                            acc_sp.at[dst_v.at[pl.ds(j * SCHUNK, SCHUNK)]],
                            add=True)

        nblk = EPT // IBLK
        ng = IBLK // SCHUNK // 2
        for p in range(passes):
            t0 = (c * passes + p) * NP
            pltpu.sync_copy(g_hbm.at[pl.ds(t0 + r0, ROWS_PER_TILE)],
                            tab_sp.at[pl.ds(r0, ROWS_PER_TILE)])
            pltpu.sync_copy(tab_sp.at[pl.ds(r0, ROWS_PER_TILE)],
                            acc_sp.at[pl.ds(r0, ROWS_PER_TILE)])
            plsc.subcore_barrier()

            def blk_body(blk, carry):
                e0 = pl.multiple_of(s * EPT + blk * IBLK, 8)
                pltpu.sync_copy(src_hbm.at[pl.ds(e0, IBLK)], src_v)
                pltpu.sync_copy(dst_hbm.at[pl.ds(e0, IBLK)], dst_v)
                start_gather(0, 0)

                def pair(g, carry2):
                    j = g * 2
                    wait_gather(0)
                    start_gather(j + 1, 1)
                    scatter(j, 0)

                    @pl.when(g < ng - 1)
                    def _():
                        start_gather(j + 2, 0)

                    wait_gather(1)
                    scatter(j + 1, 1)
                    return carry2

                lax.fori_loop(0, ng, pair, 0)
                return carry

            lax.fori_loop(0, nblk, blk_body, 0)
            plsc.subcore_barrier()
            pltpu.sync_copy(acc_sp.at[pl.ds(r0, ROWS_PER_TILE)],
                            out_hbm.at[pl.ds(t0 + r0, ROWS_PER_TILE)])
            plsc.subcore_barrier()

    return spmm


def _prep_body(deg_ref, x_ref, dinv_ref, g_ref):
    deg = deg_ref[0, :, 0:1] + deg_ref[1, :, 0:1] + 1.0
    dinv = lax.rsqrt(deg)
    dinv_ref[...] = dinv
    g = x_ref[...] * dinv
    g_ref[0] = g[:, 0:64]
    g_ref[1] = g[:, 64:128]


def _prep(deg, x):
    return pl.pallas_call(
        _prep_body,
        grid=(RB,),
        in_specs=[
            pl.BlockSpec((2, 128, 16), lambda r: (0, r, 0)),
            pl.BlockSpec((128, 128), lambda r: (r, 0)),
        ],
        out_specs=[
            pl.BlockSpec((128, 1), lambda r: (r, 0)),
            pl.BlockSpec((2, 128, 64), lambda r: (0, r, 0)),
        ],
        out_shape=[
            jax.ShapeDtypeStruct((NP, 1), jnp.float32),
            jax.ShapeDtypeStruct((2, NP, 64), jnp.float32),
        ],
        compiler_params=pltpu.CompilerParams(
            dimension_semantics=("arbitrary",)),
    )(deg, x)


def _make_layer(d_in, d_out):
    pi = d_in // DH
    po = d_out // DH

    def body(s_ref, dinv_ref, w_ref, b_ref, out_ref):
        t = jnp.dot(s_ref[0], w_ref[0:DH, :],
                    preferred_element_type=jnp.float32)
        for q in range(1, pi):
            t = t + jnp.dot(s_ref[q], w_ref[q * DH:(q + 1) * DH, :],
                            preferred_element_type=jnp.float32)
        act = jnp.maximum(dinv_ref[...] * t + b_ref[...], 0.0)
        g = act * dinv_ref[...]
        rid = lax.broadcasted_iota(jnp.int32, (128, 1), 0) + pl.program_id(0) * 128
        g = jnp.where(rid < N, g, 0.0)
        for q in range(po):
            out_ref[q] = g[:, q * DH:(q + 1) * DH]

    def layer(s, dinv, w, b):
        return pl.pallas_call(
            body,
            grid=(RB,),
            in_specs=[
                pl.BlockSpec((pi, 128, DH), lambda r: (0, r, 0)),
                pl.BlockSpec((128, 1), lambda r: (r, 0)),
                pl.BlockSpec((d_in, d_out), lambda r: (0, 0)),
                pl.BlockSpec((1, d_out), lambda r: (0, 0)),
            ],
            out_specs=pl.BlockSpec((po, 128, DH), lambda r: (0, r, 0)),
            out_shape=jax.ShapeDtypeStruct((po, NP, DH), jnp.float32),
            compiler_params=pltpu.CompilerParams(
                dimension_semantics=("arbitrary",)),
        )(s, dinv, w, b)

    return layer


def _pool_body(s_ref, dinv_ref, batch_ref, w_ref, b_ref, wfc_ref, bfc_ref,
               out_ref, acc_ref):
    r = pl.program_id(0)

    @pl.when(r == 0)
    def _init():
        acc_ref[...] = jnp.full((G, 512), -jnp.inf, jnp.float32)

    t = jnp.dot(s_ref[0], w_ref[0:64, :], preferred_element_type=jnp.float32)
    for q in range(1, 4):
        t = t + jnp.dot(s_ref[q], w_ref[q * 64:(q + 1) * 64, :],
                        preferred_element_type=jnp.float32)
    h = jnp.maximum(dinv_ref[...] * t + b_ref[...], 0.0)
    rid = lax.broadcasted_iota(jnp.int32, (128, 1), 0) + r * 128
    hm = jnp.where(rid < N, h, -jnp.inf)
    g_lo = jnp.clip(batch_ref[0, 0], 0, G - 1)
    g_hi = jnp.clip(batch_ref[127, 0], g_lo, G - 1)
    gids = lax.broadcasted_iota(jnp.int32, (G, 1), 0)

    def body(g, carry):
        sel = jnp.where(batch_ref[...] == g, hm, -jnp.inf)
        colmax = jnp.max(sel, axis=0, keepdims=True)
        acc_ref[...] = jnp.where(gids == g,
                                 jnp.maximum(acc_ref[...], colmax),
                                 acc_ref[...])
        return carry

    lax.fori_loop(g_lo, g_hi + 1, body, 0)

    @pl.when(r == RB - 1)
    def _fin():
        out_ref[...] = jnp.dot(acc_ref[...], wfc_ref[...],
                               preferred_element_type=jnp.float32) + bfc_ref[...]


def _pool(s3, dinv, batch2, w3, b3, wfc, bfc):
    return pl.pallas_call(
        _pool_body,
        grid=(RB,),
        in_specs=[
            pl.BlockSpec((4, 128, 64), lambda r: (0, r, 0)),
            pl.BlockSpec((128, 1), lambda r: (r, 0)),
            pl.BlockSpec((128, 1), lambda r: (r, 0)),
            pl.BlockSpec((256, 512), lambda r: (0, 0)),
            pl.BlockSpec((1, 512), lambda r: (0, 0)),
            pl.BlockSpec((512, 128), lambda r: (0, 0)),
            pl.BlockSpec((1, 128), lambda r: (0, 0)),
        ],
        out_specs=pl.BlockSpec((G, 128), lambda r: (0, 0)),
        out_shape=jax.ShapeDtypeStruct((G, 128), jnp.float32),
        scratch_shapes=[pltpu.VMEM((G, 512), jnp.float32)],
        compiler_params=pltpu.CompilerParams(
            dimension_semantics=("arbitrary",)),
    )(s3, dinv, batch2, w3, b3, wfc, bfc)


_degree = _degree_fn()
_spmm_p1 = _make_spmm(1)
_spmm_p2 = _make_spmm(2)
_layer1 = _make_layer(128, 128)
_layer2 = _make_layer(128, 256)


def kernel(x, edge_index, batch, W1, b1, W2, b2, W3, b3, Wfc, bfc):
    src = edge_index[0].astype(jnp.int32)
    dst = edge_index[1].astype(jnp.int32)
    srcp = jnp.concatenate([src, jnp.full((EPAD - E,), N, jnp.int32)])
    dstp = jnp.concatenate([dst, jnp.zeros((EPAD - E,), jnp.int32)])
    zeros16 = jnp.zeros((NP, 16), jnp.float32)
    ones16 = jnp.ones((CHUNK, 16), jnp.float32)
    x_pad = jnp.concatenate(
        [x, jnp.zeros((NP - N, x.shape[1]), jnp.float32)])
    batch2 = jnp.concatenate(
        [batch.astype(jnp.int32),
         jnp.full((NP - N,), G - 1, jnp.int32)]).reshape(NP, 1)

    deg = _degree(dst, zeros16, ones16)
    dinv, g1 = _prep(deg, x_pad)
    s1 = _spmm_p1(g1.reshape(2 * NP, DH), srcp, dstp).reshape(2, NP, DH)
    g2 = _layer1(s1, dinv, W1, b1.reshape(1, -1))
    s2 = _spmm_p1(g2.reshape(2 * NP, DH), srcp, dstp).reshape(2, NP, DH)
    g3 = _layer2(s2, dinv, W2, b2.reshape(1, -1))
    s3 = _spmm_p2(g3.reshape(4 * NP, DH), srcp, dstp).reshape(4, NP, DH)
    out = _pool(s3, dinv, batch2, W3, b3.reshape(1, -1),
                Wfc, bfc.reshape(1, -1))
    return out


# speedup vs baseline: 1.1754x; 1.0631x over previous
import functools

import jax
import jax.numpy as jnp
from jax import lax
from jax.experimental import pallas as pl
from jax.experimental.pallas import tpu as pltpu
from jax.experimental.pallas import tpu_sc as plsc

N = 10000
NP = 10240
E = 320000
G = 64
NC = 2
NS = 16
ROWS_PER_TILE = NP // NS
CHUNK = 128
RB = NP // 128


def _sc_mesh():
    return plsc.VectorSubcoreMesh(
        core_axis_name="c", subcore_axis_name="s",
        num_cores=NC, num_subcores=NS)


_EPT32 = E // (NC * NS)
_DEG_CHUNKS = _EPT32 // CHUNK
_DEG_TAIL = _EPT32 - _DEG_CHUNKS * CHUNK


def _degree_fn():
    @functools.partial(
        pl.kernel,
        out_type=jax.ShapeDtypeStruct((NC, NP, 16), jnp.float32),
        mesh=_sc_mesh(),
        compiler_params=pltpu.CompilerParams(use_tc_tiling_on_sc=False),
        scratch_types=[
            pltpu.VMEM_SHARED((NP, 16), jnp.float32),
            pltpu.VMEM((CHUNK, 16), jnp.float32),
            pltpu.VMEM((CHUNK,), jnp.int32),
            pltpu.VMEM((_DEG_TAIL, 16), jnp.float32),
            pltpu.VMEM((_DEG_TAIL,), jnp.int32),
        ],
    )
    def deg_kernel(dst_hbm, zeros_hbm, ones_hbm, out_hbm,
                   deg_sp, ones_v, idx_v, ones_t, idx_t):
        c = lax.axis_index("c")
        s = lax.axis_index("s")
        r0 = s * ROWS_PER_TILE
        pltpu.sync_copy(zeros_hbm.at[pl.ds(r0, ROWS_PER_TILE)],
                        deg_sp.at[pl.ds(r0, ROWS_PER_TILE)])
        pltpu.sync_copy(ones_hbm, ones_v)
        pltpu.sync_copy(ones_hbm.at[pl.ds(0, _DEG_TAIL)], ones_t)
        plsc.subcore_barrier()
        base = (c * NS + s) * _EPT32

        def chunk(j, carry):
            off = pl.multiple_of(base + j * CHUNK, 8)
            pltpu.sync_copy(dst_hbm.at[pl.ds(off, CHUNK)], idx_v)
            pltpu.sync_copy(ones_v, deg_sp.at[idx_v], add=True)
            return carry

        lax.fori_loop(0, _DEG_CHUNKS, chunk, 0)
        toff = pl.multiple_of(base + _DEG_CHUNKS * CHUNK, 8)
        pltpu.sync_copy(dst_hbm.at[pl.ds(toff, _DEG_TAIL)], idx_t)
        pltpu.sync_copy(ones_t, deg_sp.at[idx_t], add=True)
        plsc.subcore_barrier()
        pltpu.sync_copy(deg_sp.at[pl.ds(r0, ROWS_PER_TILE)],
                        out_hbm.at[c, pl.ds(r0, ROWS_PER_TILE)])

    return deg_kernel


EPT = 20480
EPAD = NS * EPT
DH = 64
SCHUNK = 128
IBLK = 5120
DEPTH = 4


def _make_spmm(passes):
    nchunk = EPT // SCHUNK

    @functools.partial(
        pl.kernel,
        out_type=jax.ShapeDtypeStruct((NC * passes * NP, DH), jnp.float32),
        mesh=_sc_mesh(),
        compiler_params=pltpu.CompilerParams(use_tc_tiling_on_sc=False),
        scratch_types=[
            pltpu.VMEM_SHARED((NP, DH), jnp.float32),
            pltpu.VMEM_SHARED((NP, DH), jnp.float32),
            pltpu.VMEM((IBLK,), jnp.int32),
            pltpu.VMEM((IBLK,), jnp.int32),
        ] + [pltpu.VMEM((SCHUNK, DH), jnp.float32)] * DEPTH
          + [pltpu.SemaphoreType.DMA] * (2 * DEPTH),
    )
    def spmm(g_hbm, src_hbm, dst_hbm, out_hbm,
             tab_sp, acc_sp, src_v, dst_v, *bufs):
        rows = bufs[0:DEPTH]
        gsem = bufs[DEPTH:2 * DEPTH]
        ssem = bufs[2 * DEPTH:3 * DEPTH]
        c = lax.axis_index("c")
        s = lax.axis_index("s")
        r0 = s * ROWS_PER_TILE
        AH = DEPTH // 2

        def start_gather(j, b):
            pltpu.async_copy(tab_sp.at[src_v.at[pl.ds(j * SCHUNK, SCHUNK)]],
                             rows[b], gsem[b])

        def wait_gather(b):
            pltpu.make_async_copy(
                tab_sp.at[pl.ds(0, SCHUNK)], rows[b], gsem[b]).wait()

        def start_scatter(j, b):
            pltpu.async_copy(rows[b],
                             acc_sp.at[dst_v.at[pl.ds(j * SCHUNK, SCHUNK)]],
                             ssem[b], add=True)

        def wait_scatter(b):
            pltpu.make_async_copy(
                rows[b], acc_sp.at[pl.ds(0, SCHUNK)], ssem[b]).wait()

        nblk = EPT // IBLK
        ng = IBLK // SCHUNK // DEPTH
        for p in range(passes):
            t0 = (c * passes + p) * NP
            pltpu.sync_copy(g_hbm.at[pl.ds(t0 + r0, ROWS_PER_TILE)],
                            tab_sp.at[pl.ds(r0, ROWS_PER_TILE)])
            pltpu.sync_copy(tab_sp.at[pl.ds(r0, ROWS_PER_TILE)],
                            acc_sp.at[pl.ds(r0, ROWS_PER_TILE)])
            plsc.subcore_barrier()

            def blk_body(blk, carry):
                e0 = pl.multiple_of(s * EPT + blk * IBLK, 8)
                pltpu.sync_copy(src_hbm.at[pl.ds(e0, IBLK)], src_v)
                pltpu.sync_copy(dst_hbm.at[pl.ds(e0, IBLK)], dst_v)
                for j in range(AH):
                    start_gather(j, j)

                def group(g, carry2):
                    for b in range(DEPTH):
                        j = g * DEPTH + b
                        wait_gather(b)
                        start_scatter(j, b)
                        bp = (b + AH) % DEPTH
                        if b < AH:
                            @pl.when(g >= 1)
                            def _():
                                wait_scatter(bp)
                            start_gather(j + AH, bp)
                        else:
                            @pl.when(g < ng - 1)
                            def _():
                                wait_scatter(bp)
                                start_gather(j + AH, bp)
                    return carry2

                lax.fori_loop(0, ng, group, 0)
                for b in range(DEPTH):
                    wait_scatter(b)
                return carry

            lax.fori_loop(0, nblk, blk_body, 0)
            plsc.subcore_barrier()
            pltpu.sync_copy(acc_sp.at[pl.ds(r0, ROWS_PER_TILE)],
                            out_hbm.at[pl.ds(t0 + r0, ROWS_PER_TILE)])
            plsc.subcore_barrier()

    return spmm


def _prep_body(deg_ref, x_ref, dinv_ref, g_ref):
    deg = deg_ref[0, :, 0:1] + deg_ref[1, :, 0:1] + 1.0
    dinv = lax.rsqrt(deg)
    dinv_ref[...] = dinv
    g = x_ref[...] * dinv
    g_ref[0] = g[:, 0:64]
    g_ref[1] = g[:, 64:128]


def _prep(deg, x):
    return pl.pallas_call(
        _prep_body,
        grid=(RB,),
        in_specs=[
            pl.BlockSpec((2, 128, 16), lambda r: (0, r, 0)),
            pl.BlockSpec((128, 128), lambda r: (r, 0)),
        ],
        out_specs=[
            pl.BlockSpec((128, 1), lambda r: (r, 0)),
            pl.BlockSpec((2, 128, 64), lambda r: (0, r, 0)),
        ],
        out_shape=[
            jax.ShapeDtypeStruct((NP, 1), jnp.float32),
            jax.ShapeDtypeStruct((2, NP, 64), jnp.float32),
        ],
        compiler_params=pltpu.CompilerParams(
            dimension_semantics=("arbitrary",)),
    )(deg, x)


def _make_layer(d_in, d_out):
    pi = d_in // DH
    po = d_out // DH

    def body(s_ref, dinv_ref, w_ref, b_ref, out_ref):
        t = jnp.dot(s_ref[0], w_ref[0:DH, :],
                    preferred_element_type=jnp.float32)
        for q in range(1, pi):
            t = t + jnp.dot(s_ref[q], w_ref[q * DH:(q + 1) * DH, :],
                            preferred_element_type=jnp.float32)
        act = jnp.maximum(dinv_ref[...] * t + b_ref[...], 0.0)
        g = act * dinv_ref[...]
        rid = lax.broadcasted_iota(jnp.int32, (128, 1), 0) + pl.program_id(0) * 128
        g = jnp.where(rid < N, g, 0.0)
        for q in range(po):
            out_ref[q] = g[:, q * DH:(q + 1) * DH]

    def layer(s, dinv, w, b):
        return pl.pallas_call(
            body,
            grid=(RB,),
            in_specs=[
                pl.BlockSpec((pi, 128, DH), lambda r: (0, r, 0)),
                pl.BlockSpec((128, 1), lambda r: (r, 0)),
                pl.BlockSpec((d_in, d_out), lambda r: (0, 0)),
                pl.BlockSpec((1, d_out), lambda r: (0, 0)),
            ],
            out_specs=pl.BlockSpec((po, 128, DH), lambda r: (0, r, 0)),
            out_shape=jax.ShapeDtypeStruct((po, NP, DH), jnp.float32),
            compiler_params=pltpu.CompilerParams(
                dimension_semantics=("arbitrary",)),
        )(s, dinv, w, b)

    return layer


def _pool_body(s_ref, dinv_ref, batch_ref, w_ref, b_ref, wfc_ref, bfc_ref,
               out_ref, acc_ref):
    r = pl.program_id(0)

    @pl.when(r == 0)
    def _init():
        acc_ref[...] = jnp.full((G, 512), -jnp.inf, jnp.float32)

    t = jnp.dot(s_ref[0], w_ref[0:64, :], preferred_element_type=jnp.float32)
    for q in range(1, 4):
        t = t + jnp.dot(s_ref[q], w_ref[q * 64:(q + 1) * 64, :],
                        preferred_element_type=jnp.float32)
    h = jnp.maximum(dinv_ref[...] * t + b_ref[...], 0.0)
    rid = lax.broadcasted_iota(jnp.int32, (128, 1), 0) + r * 128
    hm = jnp.where(rid < N, h, -jnp.inf)
    g_lo = jnp.clip(batch_ref[0, 0], 0, G - 1)
    g_hi = jnp.clip(batch_ref[127, 0], g_lo, G - 1)
    gids = lax.broadcasted_iota(jnp.int32, (G, 1), 0)

    def body(g, carry):
        sel = jnp.where(batch_ref[...] == g, hm, -jnp.inf)
        colmax = jnp.max(sel, axis=0, keepdims=True)
        acc_ref[...] = jnp.where(gids == g,
                                 jnp.maximum(acc_ref[...], colmax),
                                 acc_ref[...])
        return carry

    lax.fori_loop(g_lo, g_hi + 1, body, 0)

    @pl.when(r == RB - 1)
    def _fin():
        out_ref[...] = jnp.dot(acc_ref[...], wfc_ref[...],
                               preferred_element_type=jnp.float32) + bfc_ref[...]


def _pool(s3, dinv, batch2, w3, b3, wfc, bfc):
    return pl.pallas_call(
        _pool_body,
        grid=(RB,),
        in_specs=[
            pl.BlockSpec((4, 128, 64), lambda r: (0, r, 0)),
            pl.BlockSpec((128, 1), lambda r: (r, 0)),
            pl.BlockSpec((128, 1), lambda r: (r, 0)),
            pl.BlockSpec((256, 512), lambda r: (0, 0)),
            pl.BlockSpec((1, 512), lambda r: (0, 0)),
            pl.BlockSpec((512, 128), lambda r: (0, 0)),
            pl.BlockSpec((1, 128), lambda r: (0, 0)),
        ],
        out_specs=pl.BlockSpec((G, 128), lambda r: (0, 0)),
        out_shape=jax.ShapeDtypeStruct((G, 128), jnp.float32),
        scratch_shapes=[pltpu.VMEM((G, 512), jnp.float32)],
        compiler_params=pltpu.CompilerParams(
            dimension_semantics=("arbitrary",)),
    )(s3, dinv, batch2, w3, b3, wfc, bfc)


_degree = _degree_fn()
_spmm_p1 = _make_spmm(1)
_spmm_p2 = _make_spmm(2)
_layer1 = _make_layer(128, 128)
_layer2 = _make_layer(128, 256)


def kernel(x, edge_index, batch, W1, b1, W2, b2, W3, b3, Wfc, bfc):
    src = edge_index[0].astype(jnp.int32)
    dst = edge_index[1].astype(jnp.int32)
    srcp = jnp.concatenate([src, jnp.full((EPAD - E,), N, jnp.int32)])
    dstp = jnp.concatenate([dst, jnp.zeros((EPAD - E,), jnp.int32)])
    zeros16 = jnp.zeros((NP, 16), jnp.float32)
    ones16 = jnp.ones((CHUNK, 16), jnp.float32)
    x_pad = jnp.concatenate(
        [x, jnp.zeros((NP - N, x.shape[1]), jnp.float32)])
    batch2 = jnp.concatenate(
        [batch.astype(jnp.int32),
         jnp.full((NP - N,), G - 1, jnp.int32)]).reshape(NP, 1)

    deg = _degree(dst, zeros16, ones16)
    dinv, g1 = _prep(deg, x_pad)
    s1 = _spmm_p1(g1.reshape(2 * NP, DH), srcp, dstp).reshape(2, NP, DH)
    g2 = _layer1(s1, dinv, W1, b1.reshape(1, -1))
    s2 = _spmm_p1(g2.reshape(2 * NP, DH), srcp, dstp).reshape(2, NP, DH)
    g3 = _layer2(s2, dinv, W2, b2.reshape(1, -1))
    s3 = _spmm_p2(g3.reshape(4 * NP, DH), srcp, dstp).reshape(4, NP, DH)
    out = _pool(s3, dinv, batch2, W3, b3.reshape(1, -1),
                Wfc, bfc.reshape(1, -1))
    return out
